# baseline (speedup 1.0000x reference)
import jax
import jax.numpy as jnp
from jax import lax
from jax.experimental import pallas as pl
from jax.experimental.pallas import tpu as pltpu

N = 8192
E = 8
D_IN = 1024
D_OUT = 1024
TM = 512


def _moe_tile(g_ref, x_ref, wc_ref, b_ref, out_ref, z_ref):
    g = g_ref[...]

    neg_inf = jnp.float32(-jnp.inf)
    m1 = jnp.full((TM, 1), neg_inf, jnp.float32)
    i1 = jnp.zeros((TM, 1), jnp.int32)
    for e in range(E):
        ge = g[:, e : e + 1]
        better = ge > m1
        m1 = jnp.where(better, ge, m1)
        i1 = jnp.where(better, e, i1)
    m2 = jnp.full((TM, 1), neg_inf, jnp.float32)
    i2 = jnp.zeros((TM, 1), jnp.int32)
    for e in range(E):
        ge = jnp.where(i1 == e, neg_inf, g[:, e : e + 1])
        better = ge > m2
        m2 = jnp.where(better, ge, m2)
        i2 = jnp.where(better, e, i2)
    p1 = 1.0 / (1.0 + jnp.exp(m2 - m1))
    p2 = 1.0 - p1

    x = x_ref[...]
    ses = []
    for e in range(E):
        ses.append(jnp.where(i1 == e, p1, 0.0) + jnp.where(i2 == e, p2, 0.0))
    s = jnp.concatenate(ses, axis=1)
    acc = lax.dot_general(
        s, b_ref[...], (((1,), (0,)), ((), ())),
        preferred_element_type=jnp.float32,
    )
    for p in range(E // 2):
        zb = p % 2
        z_ref[zb, :, :D_IN] = x * ses[2 * p].astype(jnp.bfloat16)
        z_ref[zb, :, D_IN:] = x * ses[2 * p + 1].astype(jnp.bfloat16)
        acc += lax.dot_general(
            z_ref[zb],
            wc_ref[p],
            (((1,), (0,)), ((), ())),
            preferred_element_type=jnp.float32,
        )
    out_ref[...] = acc


@jax.jit
def kernel(X, G, W, b):
    Xb = X.astype(jnp.bfloat16)
    Wc = jnp.transpose(W, (0, 2, 1)).reshape(E // 2, 2 * D_IN, D_OUT)
    Wc = Wc.astype(jnp.bfloat16)
    grid = (N // TM,)
    return pl.pallas_call(
        _moe_tile,
        grid=grid,
        in_specs=[
            pl.BlockSpec((TM, E), lambda i: (i, 0)),
            pl.BlockSpec((TM, D_IN), lambda i: (i, 0)),
            pl.BlockSpec((E // 2, 2 * D_IN, D_OUT), lambda i: (0, 0, 0)),
            pl.BlockSpec((E, D_OUT), lambda i: (0, 0)),
        ],
        out_specs=pl.BlockSpec((TM, D_OUT), lambda i: (i, 0)),
        out_shape=jax.ShapeDtypeStruct((N, D_OUT), jnp.float32),
        scratch_shapes=[pltpu.VMEM((2, TM, 2 * D_IN), jnp.bfloat16)],
        compiler_params=pltpu.CompilerParams(
            dimension_semantics=("parallel",),
        ),
    )(G, Xb, Wc, b)


# speedup vs baseline: 1.2035x; 1.2035x over previous
import jax
import jax.numpy as jnp
from jax import lax
from jax.experimental import pallas as pl
from jax.experimental.pallas import tpu as pltpu

N = 8192
E = 8
D_IN = 1024
D_OUT = 1024
TM = 512


def _moe_tile(g_ref, x_ref, wt_ref, b_ref, out_ref):
    g = g_ref[...]

    neg_inf = jnp.float32(-jnp.inf)
    m1 = jnp.full((TM, 1), neg_inf, jnp.float32)
    i1 = jnp.zeros((TM, 1), jnp.int32)
    for e in range(E):
        ge = g[:, e : e + 1]
        better = ge > m1
        m1 = jnp.where(better, ge, m1)
        i1 = jnp.where(better, e, i1)
    m2 = jnp.full((TM, 1), neg_inf, jnp.float32)
    i2 = jnp.zeros((TM, 1), jnp.int32)
    for e in range(E):
        ge = jnp.where(i1 == e, neg_inf, g[:, e : e + 1])
        better = ge > m2
        m2 = jnp.where(better, ge, m2)
        i2 = jnp.where(better, e, i2)
    p1 = 1.0 / (1.0 + jnp.exp(m2 - m1))
    p2 = 1.0 - p1

    x = x_ref[...]
    acc = jnp.zeros((TM, D_OUT), jnp.float32)
    for e in range(E):
        se = jnp.where(i1 == e, p1, 0.0) + jnp.where(i2 == e, p2, 0.0)
        ye = lax.dot_general(
            x,
            wt_ref[e],
            (((1,), (1,)), ((), ())),
            preferred_element_type=jnp.float32,
        )
        acc += se * (ye + b_ref[e : e + 1, :])
    out_ref[...] = acc


@jax.jit
def kernel(X, G, W, b):
    Xb = X.astype(jnp.bfloat16)
    Wb = W.astype(jnp.bfloat16)
    grid = (N // TM,)
    return pl.pallas_call(
        _moe_tile,
        grid=grid,
        in_specs=[
            pl.BlockSpec((TM, E), lambda i: (i, 0)),
            pl.BlockSpec((TM, D_IN), lambda i: (i, 0)),
            pl.BlockSpec((E, D_OUT, D_IN), lambda i: (0, 0, 0)),
            pl.BlockSpec((E, D_OUT), lambda i: (0, 0)),
        ],
        out_specs=pl.BlockSpec((TM, D_OUT), lambda i: (i, 0)),
        out_shape=jax.ShapeDtypeStruct((N, D_OUT), jnp.float32),
        compiler_params=pltpu.CompilerParams(
            dimension_semantics=("parallel",),
        ),
    )(G, Xb, Wb, b)


# speedup vs baseline: 1.3071x; 1.0861x over previous
import jax
import jax.numpy as jnp
from jax import lax
from jax.experimental import pallas as pl
from jax.experimental.pallas import tpu as pltpu

N = 8192
E = 8
D_IN = 1024
D_OUT = 1024
TM = 512


def _moe_tile(g_ref, x_ref, wt_ref, b_ref, out_ref):
    g = g_ref[...]

    neg_inf = jnp.float32(-jnp.inf)
    m1 = jnp.full((TM, 1), neg_inf, jnp.float32)
    i1 = jnp.zeros((TM, 1), jnp.int32)
    for e in range(E):
        ge = g[:, e : e + 1]
        better = ge > m1
        m1 = jnp.where(better, ge, m1)
        i1 = jnp.where(better, e, i1)
    m2 = jnp.full((TM, 1), neg_inf, jnp.float32)
    i2 = jnp.zeros((TM, 1), jnp.int32)
    for e in range(E):
        ge = jnp.where(i1 == e, neg_inf, g[:, e : e + 1])
        better = ge > m2
        m2 = jnp.where(better, ge, m2)
        i2 = jnp.where(better, e, i2)
    p1 = 1.0 / (1.0 + jnp.exp(m2 - m1))
    p2 = 1.0 - p1

    x = x_ref[...].astype(jnp.bfloat16)
    acc = jnp.zeros((TM, D_OUT), jnp.float32)
    for e in range(E):
        se = jnp.where(i1 == e, p1, 0.0) + jnp.where(i2 == e, p2, 0.0)
        ye = lax.dot_general(
            x,
            wt_ref[e],
            (((1,), (1,)), ((), ())),
            preferred_element_type=jnp.float32,
        )
        acc += se * (ye + b_ref[e : e + 1, :])
    out_ref[...] = acc


@jax.jit
def kernel(X, G, W, b):
    Wb = W.astype(jnp.bfloat16)
    grid = (N // TM,)
    return pl.pallas_call(
        _moe_tile,
        grid=grid,
        in_specs=[
            pl.BlockSpec((TM, E), lambda i: (i, 0)),
            pl.BlockSpec((TM, D_IN), lambda i: (i, 0)),
            pl.BlockSpec((E, D_OUT, D_IN), lambda i: (0, 0, 0)),
            pl.BlockSpec((E, D_OUT), lambda i: (0, 0)),
        ],
        out_specs=pl.BlockSpec((TM, D_OUT), lambda i: (i, 0)),
        out_shape=jax.ShapeDtypeStruct((N, D_OUT), jnp.float32),
        compiler_params=pltpu.CompilerParams(
            dimension_semantics=("parallel",),
        ),
    )(G, X, Wb, b)


# speedup vs baseline: 1.3322x; 1.0192x over previous
import jax
import jax.numpy as jnp
from jax import lax
from jax.experimental import pallas as pl
from jax.experimental.pallas import tpu as pltpu

N = 8192
E = 8
D_IN = 1024
D_OUT = 1024
TM = 256


def _moe_tile(g_ref, x_ref, w_ref, b_ref, out_ref, wb_ref):
    @pl.when(pl.program_id(0) == 0)
    def _cast_weights():
        for e in range(E):
            wb_ref[e] = w_ref[e].astype(jnp.bfloat16)

    g = g_ref[...]

    neg_inf = jnp.float32(-jnp.inf)
    m1 = jnp.full((TM, 1), neg_inf, jnp.float32)
    i1 = jnp.zeros((TM, 1), jnp.int32)
    for e in range(E):
        ge = g[:, e : e + 1]
        better = ge > m1
        m1 = jnp.where(better, ge, m1)
        i1 = jnp.where(better, e, i1)
    m2 = jnp.full((TM, 1), neg_inf, jnp.float32)
    i2 = jnp.zeros((TM, 1), jnp.int32)
    for e in range(E):
        ge = jnp.where(i1 == e, neg_inf, g[:, e : e + 1])
        better = ge > m2
        m2 = jnp.where(better, ge, m2)
        i2 = jnp.where(better, e, i2)
    p1 = 1.0 / (1.0 + jnp.exp(m2 - m1))
    p2 = 1.0 - p1

    x = x_ref[...].astype(jnp.bfloat16)
    acc = jnp.zeros((TM, D_OUT), jnp.float32)
    for e in range(E):
        se = jnp.where(i1 == e, p1, 0.0) + jnp.where(i2 == e, p2, 0.0)
        ye = lax.dot_general(
            x,
            wb_ref[e],
            (((1,), (1,)), ((), ())),
            preferred_element_type=jnp.float32,
        )
        acc += se * (ye + b_ref[e : e + 1, :])
    out_ref[...] = acc


@jax.jit
def kernel(X, G, W, b):
    grid = (N // TM,)
    return pl.pallas_call(
        _moe_tile,
        grid=grid,
        in_specs=[
            pl.BlockSpec((TM, E), lambda i: (i, 0)),
            pl.BlockSpec((TM, D_IN), lambda i: (i, 0)),
            pl.BlockSpec((E, D_OUT, D_IN), lambda i: (0, 0, 0)),
            pl.BlockSpec((E, D_OUT), lambda i: (0, 0)),
        ],
        out_specs=pl.BlockSpec((TM, D_OUT), lambda i: (i, 0)),
        out_shape=jax.ShapeDtypeStruct((N, D_OUT), jnp.float32),
        scratch_shapes=[pltpu.VMEM((E, D_OUT, D_IN), jnp.bfloat16)],
        compiler_params=pltpu.CompilerParams(
            dimension_semantics=("arbitrary",),
        ),
    )(G, X, W, b)


# speedup vs baseline: 1.3923x; 1.0451x over previous
import jax
import jax.numpy as jnp
from jax import lax
from jax.experimental import pallas as pl
from jax.experimental.pallas import tpu as pltpu

N = 8192
E = 8
D_IN = 1024
D_OUT = 1024
TM = 512


def _moe_tile(g_ref, x_ref, w_ref, b_ref, out_ref, wb_ref):
    @pl.when(pl.program_id(0) == 0)
    def _cast_weights():
        for e in range(E):
            wb_ref[e] = w_ref[e].astype(jnp.bfloat16)

    g = g_ref[...]

    neg_inf = jnp.float32(-jnp.inf)
    m1 = jnp.full((TM, 1), neg_inf, jnp.float32)
    i1 = jnp.zeros((TM, 1), jnp.int32)
    for e in range(E):
        ge = g[:, e : e + 1]
        better = ge > m1
        m1 = jnp.where(better, ge, m1)
        i1 = jnp.where(better, e, i1)
    m2 = jnp.full((TM, 1), neg_inf, jnp.float32)
    i2 = jnp.zeros((TM, 1), jnp.int32)
    for e in range(E):
        ge = jnp.where(i1 == e, neg_inf, g[:, e : e + 1])
        better = ge > m2
        m2 = jnp.where(better, ge, m2)
        i2 = jnp.where(better, e, i2)
    p1 = 1.0 / (1.0 + jnp.exp(m2 - m1))
    p2 = 1.0 - p1

    x = x_ref[...].astype(jnp.bfloat16)
    acc = jnp.zeros((TM, D_OUT), jnp.float32)
    for e in range(E):
        se = jnp.where(i1 == e, p1, 0.0) + jnp.where(i2 == e, p2, 0.0)
        ye = lax.dot_general(
            x,
            wb_ref[e],
            (((1,), (1,)), ((), ())),
            preferred_element_type=jnp.float32,
        )
        acc += se * (ye + b_ref[e : e + 1, :])
    out_ref[...] = acc


@jax.jit
def kernel(X, G, W, b):
    grid = (N // TM,)
    return pl.pallas_call(
        _moe_tile,
        grid=grid,
        in_specs=[
            pl.BlockSpec((TM, E), lambda i: (i, 0)),
            pl.BlockSpec((TM, D_IN), lambda i: (i, 0)),
            pl.BlockSpec((E, D_OUT, D_IN), lambda i: (0, 0, 0)),
            pl.BlockSpec((E, D_OUT), lambda i: (0, 0)),
        ],
        out_specs=pl.BlockSpec((TM, D_OUT), lambda i: (i, 0)),
        out_shape=jax.ShapeDtypeStruct((N, D_OUT), jnp.float32),
        scratch_shapes=[pltpu.VMEM((E, D_OUT, D_IN), jnp.bfloat16)],
        compiler_params=pltpu.CompilerParams(
            dimension_semantics=("arbitrary",),
            vmem_limit_bytes=100 * 1024 * 1024,
        ),
    )(G, X, W, b)
